# baseline (speedup 1.0000x reference)
import functools

import jax
import jax.numpy as jnp
from jax import lax
from jax.experimental import pallas as pl
from jax.experimental.pallas import tpu as pltpu
from jax.experimental.pallas import tpu_sc as plsc

_B, _C, _D, _V = 1024, 20, 64, 100000
_NC, _NS = 2, 16
_NW = _NC * _NS
_ROWS = _B * _C
_RPW = _ROWS // _NW
_TV = 2048
_NT = (_V + _TV - 1) // _TV


def _sc_gather_body(table_hbm, idx_hbm, out_hbm, idx_v, rows_v, sem):
    wid = lax.axis_index("s") * _NC + lax.axis_index("c")
    base = wid * _RPW
    pltpu.sync_copy(idx_hbm.at[pl.ds(base, _RPW)], idx_v)
    pltpu.async_copy(table_hbm.at[idx_v], rows_v, sem).wait()
    pltpu.sync_copy(rows_v, out_hbm.at[pl.ds(base, _RPW)])


@functools.cache
def _get_sc_gather():
    return pl.kernel(
        _sc_gather_body,
        out_type=jax.ShapeDtypeStruct((_ROWS, _D), jnp.float32),
        mesh=plsc.VectorSubcoreMesh(core_axis_name="c", subcore_axis_name="s",
                                    num_cores=_NC, num_subcores=_NS),
        scratch_types=[
            pltpu.VMEM((_RPW,), jnp.int32),
            pltpu.VMEM((_RPW, _D), jnp.float32),
            pltpu.SemaphoreType.DMA,
        ],
        compiler_params=pltpu.CompilerParams(use_tc_tiling_on_sc=False),
    )


def _tc_body(emb_ref, w_ref, b_ref, out_ref, x_ref):
    @pl.when(pl.program_id(0) == 0)
    def _():
        e = emb_ref[...]
        ss = jnp.sum(e * e, axis=-1, keepdims=True)
        scale = jnp.minimum(1.0, 1.0 / jnp.maximum(jnp.sqrt(ss), 1e-7))
        x_ref[...] = jnp.mean(e * scale, axis=0)
    out_ref[...] = lax.dot_general(
        x_ref[...], w_ref[...], (((1,), (1,)), ((), ())),
        preferred_element_type=jnp.float32) + b_ref[...]


_tc_project = pl.pallas_call(
    _tc_body,
    grid=(_NT,),
    in_specs=[
        pl.BlockSpec((_C, _B, _D), lambda i: (0, 0, 0)),
        pl.BlockSpec((_TV, _D), lambda i: (i, 0)),
        pl.BlockSpec((1, _TV), lambda i: (0, i)),
    ],
    out_specs=pl.BlockSpec((_B, _TV), lambda i: (0, i)),
    out_shape=jax.ShapeDtypeStruct((_B, _V), jnp.float32),
    scratch_shapes=[pltpu.VMEM((_B, _D), jnp.float32)],
)


def kernel(inputs_, emb_table, W, b):
    idx = inputs_.T.reshape(-1).astype(jnp.int32)
    rows = _get_sc_gather()(emb_table, idx)
    emb = rows.reshape(_C, _B, _D)
    return _tc_project(emb, W, b.reshape(1, _V))
